# baseline (speedup 1.0000x reference)
import math

import jax
import jax.numpy as jnp
from jax import lax
from jax.experimental import pallas as pl
from jax.experimental.pallas import tpu as pltpu
from jax.experimental.pallas import tpu_sc as plsc

D_MODEL = 1024
SCALE = math.sqrt(D_MODEL)
LANES = 16

NC = 2
NS = 16
NW = NC * NS

B_TOTAL = 4 * 4096
B_PER_W = B_TOTAL // NW
CHUNK = 16
NCHUNK = B_PER_W // CHUNK
NBUF = 4
UNROLL = 8


def _scale_chunk(buf):
  def row_body(r, carry):
    def vec_body(j, carry2):
      base = j * (LANES * UNROLL)
      for u in range(UNROLL):
        sl = pl.ds(base + u * LANES, LANES)
        buf[r, sl] = buf[r, sl] * SCALE
      return carry2
    return lax.fori_loop(0, D_MODEL // (LANES * UNROLL), vec_body, carry)
  lax.fori_loop(0, CHUNK, row_body, 0)


def _emb_kernel(x_hbm, table_hbm, out_hbm, idx_v, bufs, gsems, ssems):
  wid = lax.axis_index("s") * NC + lax.axis_index("c")
  base = wid * B_PER_W

  pltpu.sync_copy(x_hbm.at[pl.ds(base, B_PER_W)], idx_v)

  def start_gather(c, slot):
    pltpu.make_async_copy(
        table_hbm.at[idx_v.at[pl.ds(c * CHUNK, CHUNK)]], bufs[slot],
        gsems[slot]).start()

  def wait_gather(c, slot):
    pltpu.make_async_copy(
        table_hbm.at[idx_v.at[pl.ds(c * CHUNK, CHUNK)]], bufs[slot],
        gsems[slot]).wait()

  def start_store(c, slot):
    pltpu.make_async_copy(
        bufs[slot], out_hbm.at[pl.ds(base + c * CHUNK, CHUNK)],
        ssems[slot]).start()

  def wait_store(c, slot):
    pltpu.make_async_copy(
        bufs[slot], out_hbm.at[pl.ds(base + c * CHUNK, CHUNK)],
        ssems[slot]).wait()

  start_gather(0, 0)
  start_gather(1, 1)

  for c in range(NCHUNK):
    g = c + 2
    if g < NCHUNK:
      slot_g = g % NBUF
      if g >= NBUF:
        wait_store(g - NBUF, slot_g)
      start_gather(g, slot_g)
    slot = c % NBUF
    wait_gather(c, slot)
    _scale_chunk(bufs[slot])
    start_store(c, slot)

  for c in range(NCHUNK - NBUF, NCHUNK):
    wait_store(c, c % NBUF)


@jax.jit
def kernel(x, table):
  idx = x.reshape(-1).astype(jnp.int32)
  mesh = plsc.VectorSubcoreMesh(
      core_axis_name="c", subcore_axis_name="s", num_cores=NC,
      num_subcores=NS)
  run = pl.kernel(
      _emb_kernel,
      out_type=jax.ShapeDtypeStruct((B_TOTAL, D_MODEL), jnp.float32),
      mesh=mesh,
      scratch_types=[
          pltpu.VMEM((B_PER_W,), jnp.int32),
          [pltpu.VMEM((CHUNK, D_MODEL), jnp.float32) for _ in range(NBUF)],
          [pltpu.SemaphoreType.DMA for _ in range(NBUF)],
          [pltpu.SemaphoreType.DMA for _ in range(NBUF)],
      ],
  )
  out = run(idx, table)
  return out.reshape(x.shape[0], x.shape[1], D_MODEL)
